# baseline (device time: 546849 ns/iter reference)
import jax
import jax.numpy as jnp
from jax import lax
from jax.experimental import pallas as pl
from jax.experimental.pallas import tpu as pltpu

N_DEV = 8


def kernel(A, B):
    m_per, k = A.shape
    _, n = B.shape
    m_half = m_per // 2

    a16 = A.astype(jnp.bfloat16)
    b16 = B.astype(jnp.bfloat16)

    def body(a_ref, b_ref, out_ref, comm, stage,
             send_r, recv_r, send_l, recv_l, cp_sems, cap_r, cap_l):
        my = lax.axis_index("i")
        left = lax.rem(my + N_DEV - 1, N_DEV)
        right = lax.rem(my + 1, N_DEV)

        seed = pltpu.make_async_copy(a_ref, comm.at[0], cp_sems.at[0, 0])
        seed.start()

        barrier_sem = pltpu.get_barrier_semaphore()
        for nbr in (left, right):
            pl.semaphore_signal(
                barrier_sem, inc=1,
                device_id=(nbr,), device_id_type=pl.DeviceIdType.MESH,
            )
        pl.semaphore_wait(barrier_sem, 2)
        seed.wait()

        n_tile = n // 4

        def tile_copies(orig_r, orig_l, j):
            js = pl.ds(j * n_tile, n_tile)
            cp_t = pltpu.make_async_copy(
                stage.at[pl.ds(0, m_half), js],
                out_ref.at[pl.ds(orig_r * m_per, m_half), js],
                cp_sems.at[0, j])
            cp_b = pltpu.make_async_copy(
                stage.at[pl.ds(m_half, m_half), js],
                out_ref.at[pl.ds(orig_l * m_per + m_half, m_half), js],
                cp_sems.at[1, j])
            return cp_t, cp_b

        def compute(ss, orig_r, orig_l, porig_r, porig_l, pred):
            for j in range(4):
                js = pl.ds(j * n_tile, n_tile)
                if pred is None:
                    pcp_t, pcp_b = tile_copies(porig_r, porig_l, j)
                    pcp_t.wait()
                    pcp_b.wait()
                else:
                    @pl.when(pred)
                    def _():
                        pcp_t, pcp_b = tile_copies(porig_r, porig_l, j)
                        pcp_t.wait()
                        pcp_b.wait()
                stage[:, js] = jnp.dot(
                    comm[ss], b_ref[:, js],
                    preferred_element_type=jnp.float32,
                ).astype(jnp.bfloat16)
                cp_t, cp_b = tile_copies(orig_r, orig_l, j)
                cp_t.start()
                cp_b.start()

        def hop(h, carry):
            ss = lax.rem(h, 2)
            sr = lax.rem(h + 1, 2)

            @pl.when(h >= 1)
            def _():
                pl.semaphore_wait(cap_r, 1)
                pl.semaphore_wait(cap_l, 1)

            rd_r = pltpu.make_async_remote_copy(
                src_ref=comm.at[ss, pl.ds(0, m_half)],
                dst_ref=comm.at[sr, pl.ds(0, m_half)],
                send_sem=send_r.at[ss], recv_sem=recv_r.at[sr],
                device_id=(right,), device_id_type=pl.DeviceIdType.MESH,
            )
            rd_l = pltpu.make_async_remote_copy(
                src_ref=comm.at[ss, pl.ds(m_half, m_half)],
                dst_ref=comm.at[sr, pl.ds(m_half, m_half)],
                send_sem=send_l.at[ss], recv_sem=recv_l.at[sr],
                device_id=(left,), device_id_type=pl.DeviceIdType.MESH,
            )
            rd_r.start()
            rd_l.start()

            compute(ss,
                    lax.rem(my - h + N_DEV, N_DEV),
                    lax.rem(my + h, N_DEV),
                    lax.rem(my - (h - 1) + N_DEV, N_DEV),
                    lax.rem(my + (h - 1) + N_DEV, N_DEV),
                    pred=h >= 1)

            rd_r.wait_send()
            rd_l.wait_send()

            @pl.when(h < N_DEV - 2)
            def _():
                pl.semaphore_signal(
                    cap_r, inc=1,
                    device_id=(left,), device_id_type=pl.DeviceIdType.MESH)
                pl.semaphore_signal(
                    cap_l, inc=1,
                    device_id=(right,), device_id_type=pl.DeviceIdType.MESH)

            rd_r.wait_recv()
            rd_l.wait_recv()

            return carry

        lax.fori_loop(0, N_DEV - 1, hop, 0)

        compute(1,
                lax.rem(my + 1, N_DEV),
                lax.rem(my + N_DEV - 1, N_DEV),
                lax.rem(my - (N_DEV - 2) + N_DEV, N_DEV),
                lax.rem(my + (N_DEV - 2), N_DEV),
                pred=None)
        for j in range(4):
            fcp_t, fcp_b = tile_copies(
                lax.rem(my + 1, N_DEV), lax.rem(my + N_DEV - 1, N_DEV), j)
            fcp_t.wait()
            fcp_b.wait()

    return pl.pallas_call(
        body,
        out_shape=jax.ShapeDtypeStruct((N_DEV * m_per, n), jnp.bfloat16),
        in_specs=[
            pl.BlockSpec(memory_space=pl.ANY),
            pl.BlockSpec(memory_space=pltpu.VMEM),
        ],
        out_specs=pl.BlockSpec(memory_space=pl.ANY),
        scratch_shapes=[
            pltpu.VMEM((2, m_per, k), jnp.bfloat16),
            pltpu.VMEM((m_per, n), jnp.bfloat16),
            pltpu.SemaphoreType.DMA((2,)),
            pltpu.SemaphoreType.DMA((2,)),
            pltpu.SemaphoreType.DMA((2,)),
            pltpu.SemaphoreType.DMA((2,)),
            pltpu.SemaphoreType.DMA((2, 4)),
            pltpu.SemaphoreType.REGULAR,
            pltpu.SemaphoreType.REGULAR,
        ],
        compiler_params=pltpu.CompilerParams(
            collective_id=0,
            vmem_limit_bytes=100 * 1024 * 1024,
        ),
    )(a16, b16)


# device time: 532719 ns/iter; 1.0265x vs baseline; 1.0265x over previous
import jax
import jax.numpy as jnp
from jax import lax
from jax.experimental import pallas as pl
from jax.experimental.pallas import tpu as pltpu

N_DEV = 8


def kernel(A, B):
    m_per, k = A.shape
    _, n = B.shape
    m_half = m_per // 2

    a16 = A.astype(jnp.bfloat16)
    b16 = B.astype(jnp.bfloat16)

    def body(a_ref, b_ref, out_ref, comm, stage,
             send_r, recv_r, send_l, recv_l, cp_sems, cap_r, cap_l):
        my = lax.axis_index("i")
        left = lax.rem(my + N_DEV - 1, N_DEV)
        right = lax.rem(my + 1, N_DEV)

        seed = pltpu.make_async_copy(a_ref, comm.at[0], cp_sems.at[0, 0])
        seed.start()

        barrier_sem = pltpu.get_barrier_semaphore()
        for nbr in (left, right):
            pl.semaphore_signal(
                barrier_sem, inc=1,
                device_id=(nbr,), device_id_type=pl.DeviceIdType.MESH,
            )
        pl.semaphore_wait(barrier_sem, 2)
        seed.wait()

        n_tile = n // 2

        def tile_copy(orig_r, orig_l, half, j):
            js = pl.ds(j * n_tile, n_tile)
            row0 = (orig_r * m_per if half == 0
                    else orig_l * m_per + m_half)
            return pltpu.make_async_copy(
                stage.at[pl.ds(half * m_half, m_half), js],
                out_ref.at[pl.ds(row0, m_half), js],
                cp_sems.at[half, j])

        def compute(ss, orig_r, orig_l, porig_r, porig_l, pred):
            for j in range(2):
                js = pl.ds(j * n_tile, n_tile)
                for half in range(2):
                    if pred is None:
                        tile_copy(porig_r, porig_l, half, j).wait()
                    else:
                        @pl.when(pred)
                        def _():
                            tile_copy(porig_r, porig_l, half, j).wait()
                    ms = pl.ds(half * m_half, m_half)
                    stage[ms, js] = jnp.dot(
                        comm[ss, ms, :], b_ref[:, js],
                        preferred_element_type=jnp.float32,
                    ).astype(jnp.bfloat16)
                    tile_copy(orig_r, orig_l, half, j).start()

        def hop(h, carry):
            ss = lax.rem(h, 2)
            sr = lax.rem(h + 1, 2)

            @pl.when(h >= 1)
            def _():
                pl.semaphore_wait(cap_r, 1)
                pl.semaphore_wait(cap_l, 1)

            rd_r = pltpu.make_async_remote_copy(
                src_ref=comm.at[ss, pl.ds(0, m_half)],
                dst_ref=comm.at[sr, pl.ds(0, m_half)],
                send_sem=send_r.at[ss], recv_sem=recv_r.at[sr],
                device_id=(right,), device_id_type=pl.DeviceIdType.MESH,
            )
            rd_l = pltpu.make_async_remote_copy(
                src_ref=comm.at[ss, pl.ds(m_half, m_half)],
                dst_ref=comm.at[sr, pl.ds(m_half, m_half)],
                send_sem=send_l.at[ss], recv_sem=recv_l.at[sr],
                device_id=(left,), device_id_type=pl.DeviceIdType.MESH,
            )
            rd_r.start()
            rd_l.start()

            compute(ss,
                    lax.rem(my - h + N_DEV, N_DEV),
                    lax.rem(my + h, N_DEV),
                    lax.rem(my - (h - 1) + N_DEV, N_DEV),
                    lax.rem(my + (h - 1) + N_DEV, N_DEV),
                    pred=h >= 1)

            rd_r.wait_send()
            rd_l.wait_send()

            @pl.when(h < N_DEV - 2)
            def _():
                pl.semaphore_signal(
                    cap_r, inc=1,
                    device_id=(left,), device_id_type=pl.DeviceIdType.MESH)
                pl.semaphore_signal(
                    cap_l, inc=1,
                    device_id=(right,), device_id_type=pl.DeviceIdType.MESH)

            rd_r.wait_recv()
            rd_l.wait_recv()

            return carry

        lax.fori_loop(0, N_DEV - 1, hop, 0)

        compute(1,
                lax.rem(my + 1, N_DEV),
                lax.rem(my + N_DEV - 1, N_DEV),
                lax.rem(my - (N_DEV - 2) + N_DEV, N_DEV),
                lax.rem(my + (N_DEV - 2), N_DEV),
                pred=None)
        for j in range(2):
            for half in range(2):
                tile_copy(lax.rem(my + 1, N_DEV),
                          lax.rem(my + N_DEV - 1, N_DEV), half, j).wait()

    return pl.pallas_call(
        body,
        out_shape=jax.ShapeDtypeStruct((N_DEV * m_per, n), jnp.bfloat16),
        in_specs=[
            pl.BlockSpec(memory_space=pl.ANY),
            pl.BlockSpec(memory_space=pltpu.VMEM),
        ],
        out_specs=pl.BlockSpec(memory_space=pl.ANY),
        scratch_shapes=[
            pltpu.VMEM((2, m_per, k), jnp.bfloat16),
            pltpu.VMEM((m_per, n), jnp.bfloat16),
            pltpu.SemaphoreType.DMA((2,)),
            pltpu.SemaphoreType.DMA((2,)),
            pltpu.SemaphoreType.DMA((2,)),
            pltpu.SemaphoreType.DMA((2,)),
            pltpu.SemaphoreType.DMA((2, 2)),
            pltpu.SemaphoreType.REGULAR,
            pltpu.SemaphoreType.REGULAR,
        ],
        compiler_params=pltpu.CompilerParams(
            collective_id=0,
            vmem_limit_bytes=100 * 1024 * 1024,
        ),
    )(a16, b16)


# device time: 524651 ns/iter; 1.0423x vs baseline; 1.0154x over previous
import jax
import jax.numpy as jnp
from jax import lax
from jax.experimental import pallas as pl
from jax.experimental.pallas import tpu as pltpu

N_DEV = 8


def kernel(A, B):
    m_per, k = A.shape
    _, n = B.shape
    m_half = m_per // 2

    a16 = A.astype(jnp.bfloat16)
    b16 = B.astype(jnp.bfloat16)

    def body(a_ref, b_ref, out_ref, comm, stage,
             send_r, recv_r, send_l, recv_l, cp_sems, ep_sems,
             cap_r, cap_l):
        my = lax.axis_index("i")
        left = lax.rem(my + N_DEV - 1, N_DEV)
        right = lax.rem(my + 1, N_DEV)

        seed = pltpu.make_async_copy(a_ref, comm.at[0], cp_sems.at[0, 0])
        seed.start()

        barrier_sem = pltpu.get_barrier_semaphore()
        for nbr in (left, right):
            pl.semaphore_signal(
                barrier_sem, inc=1,
                device_id=(nbr,), device_id_type=pl.DeviceIdType.MESH,
            )
        pl.semaphore_wait(barrier_sem, 2)
        seed.wait()

        n_tile = n // 2

        def tile_copy(orig_r, orig_l, half, j):
            js = pl.ds(j * n_tile, n_tile)
            row0 = (orig_r * m_per if half == 0
                    else orig_l * m_per + m_half)
            return pltpu.make_async_copy(
                stage.at[pl.ds(half * m_half, m_half), js],
                out_ref.at[pl.ds(row0, m_half), js],
                cp_sems.at[half, j])

        def compute(ss, orig_r, orig_l, porig_r, porig_l, pred):
            for j in range(2):
                js = pl.ds(j * n_tile, n_tile)
                for half in range(2):
                    if pred is None:
                        tile_copy(porig_r, porig_l, half, j).wait()
                    else:
                        @pl.when(pred)
                        def _():
                            tile_copy(porig_r, porig_l, half, j).wait()
                    ms = pl.ds(half * m_half, m_half)
                    stage[ms, js] = jnp.dot(
                        comm[ss, ms, :], b_ref[:, js],
                        preferred_element_type=jnp.float32,
                    ).astype(jnp.bfloat16)
                    tile_copy(orig_r, orig_l, half, j).start()

        def hop(h, carry):
            ss = lax.rem(h, 2)
            sr = lax.rem(h + 1, 2)

            @pl.when(h >= 1)
            def _():
                pl.semaphore_wait(cap_r, 1)
                pl.semaphore_wait(cap_l, 1)

            rd_r = pltpu.make_async_remote_copy(
                src_ref=comm.at[ss, pl.ds(0, m_half)],
                dst_ref=comm.at[sr, pl.ds(0, m_half)],
                send_sem=send_r.at[ss], recv_sem=recv_r.at[sr],
                device_id=(right,), device_id_type=pl.DeviceIdType.MESH,
            )
            rd_l = pltpu.make_async_remote_copy(
                src_ref=comm.at[ss, pl.ds(m_half, m_half)],
                dst_ref=comm.at[sr, pl.ds(m_half, m_half)],
                send_sem=send_l.at[ss], recv_sem=recv_l.at[sr],
                device_id=(left,), device_id_type=pl.DeviceIdType.MESH,
            )
            rd_r.start()
            rd_l.start()

            compute(ss,
                    lax.rem(my - h + N_DEV, N_DEV),
                    lax.rem(my + h, N_DEV),
                    lax.rem(my - (h - 1) + N_DEV, N_DEV),
                    lax.rem(my + (h - 1) + N_DEV, N_DEV),
                    pred=h >= 1)

            rd_r.wait_send()
            rd_l.wait_send()

            @pl.when(h < N_DEV - 2)
            def _():
                pl.semaphore_signal(
                    cap_r, inc=1,
                    device_id=(left,), device_id_type=pl.DeviceIdType.MESH)
                pl.semaphore_signal(
                    cap_l, inc=1,
                    device_id=(right,), device_id_type=pl.DeviceIdType.MESH)

            rd_r.wait_recv()
            rd_l.wait_recv()

            return carry

        lax.fori_loop(0, N_DEV - 2, hop, 0)

        pl.semaphore_wait(cap_r, 1)
        pl.semaphore_wait(cap_l, 1)

        m_q = m_half // 2

        def sub_rdma(q):
            rows = pl.ds(q * m_q, m_q)
            if q < 2:
                return pltpu.make_async_remote_copy(
                    src_ref=comm.at[0, rows], dst_ref=comm.at[1, rows],
                    send_sem=send_r.at[q], recv_sem=recv_r.at[1 - q],
                    device_id=(right,),
                    device_id_type=pl.DeviceIdType.MESH)
            return pltpu.make_async_remote_copy(
                src_ref=comm.at[0, rows], dst_ref=comm.at[1, rows],
                send_sem=send_l.at[q - 2], recv_sem=recv_l.at[3 - q],
                device_id=(left,), device_id_type=pl.DeviceIdType.MESH)

        subs = [sub_rdma(q) for q in range(4)]
        for s in subs:
            s.start()

        compute(0,
                lax.rem(my - (N_DEV - 2) + N_DEV, N_DEV),
                lax.rem(my + (N_DEV - 2), N_DEV),
                lax.rem(my - (N_DEV - 3) + N_DEV, N_DEV),
                lax.rem(my + (N_DEV - 3), N_DEV),
                pred=None)
        for s in subs:
            s.wait_send()

        for j in range(2):
            for half in range(2):
                tile_copy(lax.rem(my - (N_DEV - 2) + N_DEV, N_DEV),
                          lax.rem(my + (N_DEV - 2), N_DEV), half, j).wait()

        orig_top = lax.rem(my + 1, N_DEV)
        orig_bot = lax.rem(my + N_DEV - 1, N_DEV)

        def ep_quarter(q):
            rows = pl.ds(q * m_q, m_q)
            orig = orig_top if q < 2 else orig_bot
            out_rows = pl.ds(orig * m_per + q * m_q, m_q)
            stage[rows, :] = jnp.dot(
                comm[1, rows, :], b_ref[:, :],
                preferred_element_type=jnp.float32,
            ).astype(jnp.bfloat16)
            return pltpu.make_async_copy(
                stage.at[rows, :], out_ref.at[out_rows, :], ep_sems.at[q])

        for q in (0, 2, 1, 3):
            subs[q].wait_recv()
            ep_quarter(q).start()
        for q in range(4):
            ep_quarter_wait = pltpu.make_async_copy(
                stage.at[pl.ds(q * m_q, m_q), :],
                out_ref.at[pl.ds(
                    (orig_top if q < 2 else orig_bot) * m_per + q * m_q,
                    m_q), :],
                ep_sems.at[q])
            ep_quarter_wait.wait()

    return pl.pallas_call(
        body,
        out_shape=jax.ShapeDtypeStruct((N_DEV * m_per, n), jnp.bfloat16),
        in_specs=[
            pl.BlockSpec(memory_space=pl.ANY),
            pl.BlockSpec(memory_space=pltpu.VMEM),
        ],
        out_specs=pl.BlockSpec(memory_space=pl.ANY),
        scratch_shapes=[
            pltpu.VMEM((2, m_per, k), jnp.bfloat16),
            pltpu.VMEM((m_per, n), jnp.bfloat16),
            pltpu.SemaphoreType.DMA((2,)),
            pltpu.SemaphoreType.DMA((2,)),
            pltpu.SemaphoreType.DMA((2,)),
            pltpu.SemaphoreType.DMA((2,)),
            pltpu.SemaphoreType.DMA((2, 2)),
            pltpu.SemaphoreType.DMA((4,)),
            pltpu.SemaphoreType.REGULAR,
            pltpu.SemaphoreType.REGULAR,
        ],
        compiler_params=pltpu.CompilerParams(
            collective_id=0,
            vmem_limit_bytes=100 * 1024 * 1024,
        ),
    )(a16, b16)
